# baseline (speedup 1.0000x reference)
import functools

import jax
import jax.numpy as jnp
from jax import lax
from jax.experimental import pallas as pl
from jax.experimental.pallas import tpu as pltpu
from jax.experimental.pallas import tpu_sc as plsc

N = 10000
E = 320000
D = 128

NC = 2
NS = 16
NW = NC * NS
EPT = E // NW
CHUNK = 50
G = 20
NCHUNK = EPT // CHUNK
NGROUP = NCHUNK // G
NBUF = 4
NPAD = 10240
RPT = NPAD // NS
ZROWS = 128


def _agg_body(h_hbm, src_hbm, dst_hbm, z_hbm, out_hbm,
              sidx, didx, rows, acc,
              isem0, isem1, gsem0, gsem1, gsem2, gsem3,
              ssem0, ssem1, ssem2, ssem3):
    cid = lax.axis_index("c")
    sid = lax.axis_index("s")
    wid = cid * NS + sid
    gsems = (gsem0, gsem1, gsem2, gsem3)
    ssems = (ssem0, ssem1, ssem2, ssem3)
    isems = (isem0, isem1)

    def fire_idx(slot, grp, sem):
        pltpu.async_copy(src_hbm.at[wid, grp], sidx.at[slot], sem)
        pltpu.async_copy(dst_hbm.at[wid, grp], didx.at[slot], sem)

    def wait_idx(slot, sem):
        pltpu.make_async_copy(src_hbm.at[wid, 0], sidx.at[slot], sem).wait()
        pltpu.make_async_copy(dst_hbm.at[wid, 0], didx.at[slot], sem).wait()

    def fire_gather(slot, k, buf):
        pltpu.async_copy(h_hbm.at[sidx.at[slot, k]], rows.at[buf], gsems[buf])

    def wait_gather(buf):
        pltpu.make_async_copy(h_hbm.at[sidx.at[0, 0]], rows.at[buf],
                              gsems[buf]).wait()

    def fire_scatter(slot, k, buf):
        pltpu.async_copy(rows.at[buf], acc.at[didx.at[slot, k]], ssems[buf],
                         add=True)

    def wait_scatter(buf):
        pltpu.make_async_copy(rows.at[buf], acc.at[didx.at[0, 0]],
                              ssems[buf]).wait()

    fire_idx(0, 0, isem0)
    fire_idx(1, 1, isem1)

    for k in range(RPT // ZROWS):
        pltpu.sync_copy(z_hbm, acc.at[pl.ds(sid * RPT + k * ZROWS, ZROWS)])

    wait_idx(0, isem0)
    fire_gather(0, 0, 0)
    fire_gather(0, 1, 1)
    plsc.subcore_barrier()

    def group_body(g, s, first):
        s2 = 1 - s
        for k in range(G):
            buf = k % NBUF
            buf2 = (k + 2) % NBUF
            wait_gather(buf)
            fire_scatter(s, k, buf)
            if not (first and k < 2):
                wait_scatter(buf2)
            if k == G - 2:
                wait_idx(s2, isems[s2])
            if k < G - 2:
                fire_gather(s, k + 2, buf2)
            else:
                fire_gather(s2, k - (G - 2), buf2)
        g2 = jnp.where(g + 2 >= NGROUP, g + 2 - NGROUP, g + 2)
        fire_idx(s, g2, isems[s])

    group_body(0, 0, True)
    group_body(1, 1, False)

    def outer(gp, carry):
        group_body(2 * gp + 2, 0, False)
        group_body(2 * gp + 3, 1, False)
        return carry

    lax.fori_loop(0, (NGROUP - 2) // 2, outer, 0)
    wait_gather(0)
    wait_gather(1)
    wait_scatter(2)
    wait_scatter(3)
    wait_idx(1, isem1)
    plsc.subcore_barrier()

    for k in range(RPT // ZROWS):
        r0 = sid * RPT + k * ZROWS
        pltpu.sync_copy(acc.at[pl.ds(r0, ZROWS)], out_hbm.at[cid, pl.ds(r0, ZROWS)])


_agg = pl.kernel(
    _agg_body,
    out_type=jax.ShapeDtypeStruct((NC, NPAD, D), jnp.float32),
    mesh=plsc.VectorSubcoreMesh(core_axis_name="c", subcore_axis_name="s"),
    scratch_types=[
        pltpu.VMEM((2, G, CHUNK), jnp.int32),
        pltpu.VMEM((2, G, CHUNK), jnp.int32),
        pltpu.VMEM((NBUF, CHUNK, D), jnp.float32),
        pltpu.VMEM_SHARED((NPAD, D), jnp.float32),
    ] + [pltpu.SemaphoreType.DMA] * 10,
)


BLK = 400


def _mlp_body(eps_ref, x_ref, p_ref, w1_ref, b1_ref, w2_ref, b2_ref, o_ref):
    hb = (1.0 + eps_ref[0]) * x_ref[...] + p_ref[0] + p_ref[1]
    t = jnp.dot(hb, w1_ref[...], preferred_element_type=jnp.float32) + b1_ref[...]
    t = jnp.maximum(t, 0.0)
    o_ref[...] = jnp.dot(t, w2_ref[...], preferred_element_type=jnp.float32) + b2_ref[...]


def _mlp(x, p, W1, b1, W2, b2, eps):
    return pl.pallas_call(
        _mlp_body,
        grid=(N // BLK,),
        in_specs=[
            pl.BlockSpec(memory_space=pltpu.SMEM),
            pl.BlockSpec((BLK, D), lambda i: (i, 0)),
            pl.BlockSpec((NC, BLK, D), lambda i: (0, i, 0)),
            pl.BlockSpec((D, D), lambda i: (0, 0)),
            pl.BlockSpec((1, D), lambda i: (0, 0)),
            pl.BlockSpec((D, D), lambda i: (0, 0)),
            pl.BlockSpec((1, D), lambda i: (0, 0)),
        ],
        out_specs=pl.BlockSpec((BLK, D), lambda i: (i, 0)),
        out_shape=jax.ShapeDtypeStruct((N, D), jnp.float32),
    )(eps.reshape(1), x, p, W1, b1.reshape(1, D), W2, b2.reshape(1, D))


def kernel(x, edge_index, W1_0, b1_0, W2_0, b2_0, eps_0,
           W1_1, b1_1, W2_1, b2_1, eps_1):
    h = x
    src = edge_index[0].reshape(NW, NGROUP, G, CHUNK)
    dst = edge_index[1].reshape(NW, NGROUP, G, CHUNK)
    zeros = jnp.zeros((ZROWS, D), jnp.float32)
    for (W1, b1, W2, b2, eps) in ((W1_0, b1_0, W2_0, b2_0, eps_0),
                                  (W1_1, b1_1, W2_1, b2_1, eps_1)):
        p = _agg(h, src, dst, zeros)
        h = _mlp(h, p, W1, b1, W2, b2, eps)
    return h


# speedup vs baseline: 1.3874x; 1.3874x over previous
import functools

import jax
import jax.numpy as jnp
from jax import lax
from jax.experimental import pallas as pl
from jax.experimental.pallas import tpu as pltpu
from jax.experimental.pallas import tpu_sc as plsc

N = 10000
E = 320000
D = 128

NC = 2
NS = 16
NW = NC * NS
EPT = E // NW
CHUNK = 125
G = 8
NCHUNK = EPT // CHUNK
NGROUP = NCHUNK // G
NBUF = 2
NPAD = 10240
RPT = NPAD // NS
ZROWS = 128


ZR = 120


def _agg_body(h_hbm, ei_hbm, out_hbm,
              sidx, didx, rows, acc, isem0, isem1, gsem0, gsem1):
    cid = lax.axis_index("c")
    sid = lax.axis_index("s")
    wid = cid * NS + sid
    gsems = (gsem0, gsem1)
    isems = (isem0, isem1)

    def fire_idx(slot, grp, sem):
        pltpu.async_copy(ei_hbm.at[0, wid, grp], sidx.at[slot], sem)
        pltpu.async_copy(ei_hbm.at[1, wid, grp], didx.at[slot], sem)

    def wait_idx(slot, sem):
        pltpu.make_async_copy(ei_hbm.at[0, wid, 0], sidx.at[slot], sem).wait()
        pltpu.make_async_copy(ei_hbm.at[1, wid, 0], didx.at[slot], sem).wait()

    def fire_gather(slot, k, buf):
        pltpu.async_copy(h_hbm.at[sidx.at[slot, k]], rows.at[buf], gsems[buf])

    def wait_gather(buf):
        pltpu.make_async_copy(h_hbm.at[sidx.at[0, 0]], rows.at[buf],
                              gsems[buf]).wait()

    fire_idx(0, 0, isem0)
    fire_idx(1, 1, isem1)

    def zrow(r, carry):
        for c in range(D // 16):
            rows[0, r, pl.ds(c * 16, 16)] = jnp.zeros((16,), jnp.float32)
        return carry

    lax.fori_loop(0, ZR, zrow, 0)
    for k in range(RPT // ZR):
        pltpu.sync_copy(rows.at[0, pl.ds(0, ZR)],
                        acc.at[pl.ds(sid * RPT + k * ZR, ZR)])
    pltpu.sync_copy(rows.at[0, pl.ds(0, RPT - (RPT // ZR) * ZR)],
                    acc.at[pl.ds(sid * RPT + (RPT // ZR) * ZR,
                                 RPT - (RPT // ZR) * ZR)])

    wait_idx(0, isem0)
    fire_gather(0, 0, 0)
    fire_gather(0, 1, 1)
    plsc.subcore_barrier()

    def group_body(g, s):
        s2 = 1 - s
        for k in range(G):
            buf = k % 2
            wait_gather(buf)
            pltpu.sync_copy(rows.at[buf], acc.at[didx.at[s, k]], add=True)
            if k == G - 2:
                wait_idx(s2, isems[s2])
            if k < G - 2:
                fire_gather(s, k + 2, buf)
            else:
                fire_gather(s2, k - (G - 2), buf)
        g2 = jnp.where(g + 2 >= NGROUP, g + 2 - NGROUP, g + 2)
        fire_idx(s, g2, isems[s])

    def outer(gp, carry):
        group_body(2 * gp, 0)
        group_body(2 * gp + 1, 1)
        return carry

    lax.fori_loop(0, NGROUP // 2, outer, 0)
    wait_gather(0)
    wait_gather(1)
    wait_idx(1, isem1)
    plsc.subcore_barrier()

    for k in range(RPT // ZROWS):
        r0 = sid * RPT + k * ZROWS
        pltpu.async_copy(acc.at[pl.ds(r0, ZROWS)],
                         out_hbm.at[cid, pl.ds(r0, ZROWS)], gsem0)
    for k in range(RPT // ZROWS):
        r0 = sid * RPT + k * ZROWS
        pltpu.make_async_copy(acc.at[pl.ds(r0, ZROWS)],
                              out_hbm.at[cid, pl.ds(r0, ZROWS)], gsem0).wait()


_agg = pl.kernel(
    _agg_body,
    out_type=jax.ShapeDtypeStruct((NC, NPAD, D), jnp.float32),
    mesh=plsc.VectorSubcoreMesh(core_axis_name="c", subcore_axis_name="s"),
    scratch_types=[
        pltpu.VMEM((2, G, CHUNK), jnp.int32),
        pltpu.VMEM((2, G, CHUNK), jnp.int32),
        pltpu.VMEM((NBUF, CHUNK, D), jnp.float32),
        pltpu.VMEM_SHARED((NPAD, D), jnp.float32),
        pltpu.SemaphoreType.DMA,
        pltpu.SemaphoreType.DMA,
        pltpu.SemaphoreType.DMA,
        pltpu.SemaphoreType.DMA,
    ],
)


BLK = 2000


def _mlp_body(eps_ref, x_ref, p_ref, w1_ref, b1_ref, w2_ref, b2_ref, o_ref):
    hb = (1.0 + eps_ref[0]) * x_ref[...] + p_ref[0] + p_ref[1]
    t = jnp.dot(hb, w1_ref[...], preferred_element_type=jnp.float32) + b1_ref[...]
    t = jnp.maximum(t, 0.0)
    o_ref[...] = jnp.dot(t, w2_ref[...], preferred_element_type=jnp.float32) + b2_ref[...]


def _mlp(x, p, W1, b1, W2, b2, eps):
    return pl.pallas_call(
        _mlp_body,
        grid=(N // BLK,),
        in_specs=[
            pl.BlockSpec(memory_space=pltpu.SMEM),
            pl.BlockSpec((BLK, D), lambda i: (i, 0)),
            pl.BlockSpec((NC, BLK, D), lambda i: (0, i, 0)),
            pl.BlockSpec((D, D), lambda i: (0, 0)),
            pl.BlockSpec((1, D), lambda i: (0, 0)),
            pl.BlockSpec((D, D), lambda i: (0, 0)),
            pl.BlockSpec((1, D), lambda i: (0, 0)),
        ],
        out_specs=pl.BlockSpec((BLK, D), lambda i: (i, 0)),
        out_shape=jax.ShapeDtypeStruct((N, D), jnp.float32),
    )(eps.reshape(1), x, p, W1, b1.reshape(1, D), W2, b2.reshape(1, D))


def kernel(x, edge_index, W1_0, b1_0, W2_0, b2_0, eps_0,
           W1_1, b1_1, W2_1, b2_1, eps_1):
    h = x
    ei = edge_index.reshape(2, NW, NGROUP, G, CHUNK)
    for (W1, b1, W2, b2, eps) in ((W1_0, b1_0, W2_0, b2_0, eps_0),
                                  (W1_1, b1_1, W2_1, b2_1, eps_1)):
        p = _agg(h, ei)
        h = _mlp(h, p, W1, b1, W2, b2, eps)
    return h


# speedup vs baseline: 1.3940x; 1.0047x over previous
import functools

import jax
import jax.numpy as jnp
from jax import lax
from jax.experimental import pallas as pl
from jax.experimental.pallas import tpu as pltpu
from jax.experimental.pallas import tpu_sc as plsc

N = 10000
E = 320000
D = 128

NC = 2
NS = 16
NW = NC * NS
EPT = E // NW
CHUNK = 125
G = 8
NCHUNK = EPT // CHUNK
NGROUP = NCHUNK // G
NBUF = 2
NPAD = 10240
RPT = NPAD // NS
ZROWS = 128


ZR = 120


def _agg_body(h_hbm, ei_hbm, out_hbm,
              sidx, didx, rows, acc, isem0, isem1, gsem0, gsem1, zsem):
    cid = lax.axis_index("c")
    sid = lax.axis_index("s")
    wid = cid * NS + sid
    gsems = (gsem0, gsem1)
    isems = (isem0, isem1)

    def fire_idx(slot, grp, sem):
        pltpu.async_copy(ei_hbm.at[0, wid, grp], sidx.at[slot], sem)
        pltpu.async_copy(ei_hbm.at[1, wid, grp], didx.at[slot], sem)

    def wait_idx(slot, sem):
        pltpu.make_async_copy(ei_hbm.at[0, wid, 0], sidx.at[slot], sem).wait()
        pltpu.make_async_copy(ei_hbm.at[1, wid, 0], didx.at[slot], sem).wait()

    def fire_gather(slot, k, buf):
        pltpu.async_copy(h_hbm.at[sidx.at[slot, k]], rows.at[buf], gsems[buf])

    def wait_gather(buf):
        pltpu.make_async_copy(h_hbm.at[sidx.at[0, 0]], rows.at[buf],
                              gsems[buf]).wait()

    fire_idx(0, 0, isem0)
    fire_idx(1, 1, isem1)

    def zrow(r, carry):
        for c in range(D // 16):
            rows[0, r, pl.ds(c * 16, 16)] = jnp.zeros((16,), jnp.float32)
        return carry

    lax.fori_loop(0, ZR, zrow, 0)
    zcopies = []
    for k in range(RPT // ZR):
        zcopies.append(pltpu.async_copy(
            rows.at[0, pl.ds(0, ZR)],
            acc.at[pl.ds(sid * RPT + k * ZR, ZR)], zsem))
    zcopies.append(pltpu.async_copy(
        rows.at[0, pl.ds(0, RPT - (RPT // ZR) * ZR)],
        acc.at[pl.ds(sid * RPT + (RPT // ZR) * ZR,
                     RPT - (RPT // ZR) * ZR)], zsem))

    wait_idx(0, isem0)
    fire_gather(0, 1, 1)
    for c in zcopies:
        c.wait()
    fire_gather(0, 0, 0)
    plsc.subcore_barrier()

    def group_body(g, s):
        s2 = 1 - s
        for k in range(G):
            buf = k % 2
            wait_gather(buf)
            pltpu.sync_copy(rows.at[buf], acc.at[didx.at[s, k]], add=True)
            if k == G - 2:
                wait_idx(s2, isems[s2])
            if k < G - 2:
                fire_gather(s, k + 2, buf)
            else:
                fire_gather(s2, k - (G - 2), buf)
        g2 = jnp.where(g + 2 >= NGROUP, g + 2 - NGROUP, g + 2)
        fire_idx(s, g2, isems[s])

    def outer(gp, carry):
        group_body(2 * gp, 0)
        group_body(2 * gp + 1, 1)
        return carry

    lax.fori_loop(0, NGROUP // 2, outer, 0)
    wait_gather(0)
    wait_gather(1)
    wait_idx(1, isem1)
    plsc.subcore_barrier()

    for k in range(RPT // ZROWS):
        r0 = sid * RPT + k * ZROWS
        pltpu.async_copy(acc.at[pl.ds(r0, ZROWS)],
                         out_hbm.at[cid, pl.ds(r0, ZROWS)], gsem0)
    for k in range(RPT // ZROWS):
        r0 = sid * RPT + k * ZROWS
        pltpu.make_async_copy(acc.at[pl.ds(r0, ZROWS)],
                              out_hbm.at[cid, pl.ds(r0, ZROWS)], gsem0).wait()


_agg = pl.kernel(
    _agg_body,
    out_type=jax.ShapeDtypeStruct((NC, NPAD, D), jnp.float32),
    mesh=plsc.VectorSubcoreMesh(core_axis_name="c", subcore_axis_name="s"),
    scratch_types=[
        pltpu.VMEM((2, G, CHUNK), jnp.int32),
        pltpu.VMEM((2, G, CHUNK), jnp.int32),
        pltpu.VMEM((NBUF, CHUNK, D), jnp.float32),
        pltpu.VMEM_SHARED((NPAD, D), jnp.float32),
        pltpu.SemaphoreType.DMA,
        pltpu.SemaphoreType.DMA,
        pltpu.SemaphoreType.DMA,
        pltpu.SemaphoreType.DMA,
        pltpu.SemaphoreType.DMA,
    ],
)


BLK = 2000


def _mlp_body(eps_ref, x_ref, p_ref, w1_ref, b1_ref, w2_ref, b2_ref, o_ref):
    hb = (1.0 + eps_ref[0]) * x_ref[...] + p_ref[0] + p_ref[1]
    t = jnp.dot(hb, w1_ref[...], preferred_element_type=jnp.float32) + b1_ref[...]
    t = jnp.maximum(t, 0.0)
    o_ref[...] = jnp.dot(t, w2_ref[...], preferred_element_type=jnp.float32) + b2_ref[...]


def _mlp(x, p, W1, b1, W2, b2, eps):
    return pl.pallas_call(
        _mlp_body,
        grid=(N // BLK,),
        in_specs=[
            pl.BlockSpec(memory_space=pltpu.SMEM),
            pl.BlockSpec((BLK, D), lambda i: (i, 0)),
            pl.BlockSpec((NC, BLK, D), lambda i: (0, i, 0)),
            pl.BlockSpec((D, D), lambda i: (0, 0)),
            pl.BlockSpec((1, D), lambda i: (0, 0)),
            pl.BlockSpec((D, D), lambda i: (0, 0)),
            pl.BlockSpec((1, D), lambda i: (0, 0)),
        ],
        out_specs=pl.BlockSpec((BLK, D), lambda i: (i, 0)),
        out_shape=jax.ShapeDtypeStruct((N, D), jnp.float32),
    )(eps.reshape(1), x, p, W1, b1.reshape(1, D), W2, b2.reshape(1, D))


def kernel(x, edge_index, W1_0, b1_0, W2_0, b2_0, eps_0,
           W1_1, b1_1, W2_1, b2_1, eps_1):
    h = x
    ei = edge_index.reshape(2, NW, NGROUP, G, CHUNK)
    for (W1, b1, W2, b2, eps) in ((W1_0, b1_0, W2_0, b2_0, eps_0),
                                  (W1_1, b1_1, W2_1, b2_1, eps_1)):
        p = _agg(h, ei)
        h = _mlp(h, p, W1, b1, W2, b2, eps)
    return h


# speedup vs baseline: 1.3954x; 1.0010x over previous
import jax
import jax.numpy as jnp
from jax import lax
from jax.experimental import pallas as pl
from jax.experimental.pallas import tpu as pltpu
from jax.experimental.pallas import tpu_sc as plsc

N = 10000
E = 320000
D = 128

NC = 2
NS = 16
NW = NC * NS
EPT = E // NW
CHUNK = 125
G = 8
NCHUNK = EPT // CHUNK
NGROUP = NCHUNK // G
NBUF = 2
NPAD = 10240
RPT = NPAD // NS
ZROWS = 128


ZR = 120


def _agg_body(h_hbm, ei_hbm, out_hbm,
              sidx, didx, rows, acc, isem0, isem1, gsem0, gsem1, zsem):
    cid = lax.axis_index("c")
    sid = lax.axis_index("s")
    wid = cid * NS + sid
    gsems = (gsem0, gsem1)
    isems = (isem0, isem1)

    def fire_idx(slot, grp, sem):
        pltpu.async_copy(ei_hbm.at[0, wid, grp], sidx.at[slot], sem)
        pltpu.async_copy(ei_hbm.at[1, wid, grp], didx.at[slot], sem)

    def wait_idx(slot, sem):
        pltpu.make_async_copy(ei_hbm.at[0, wid, 0], sidx.at[slot], sem).wait()
        pltpu.make_async_copy(ei_hbm.at[1, wid, 0], didx.at[slot], sem).wait()

    def fire_gather(slot, k, buf):
        pltpu.async_copy(h_hbm.at[sidx.at[slot, k]], rows.at[buf], gsems[buf])

    def wait_gather(buf):
        pltpu.make_async_copy(h_hbm.at[sidx.at[0, 0]], rows.at[buf],
                              gsems[buf]).wait()

    fire_idx(0, 0, isem0)
    fire_idx(1, 1, isem1)

    def zrow(r, carry):
        for c in range(D // 16):
            rows[0, r, pl.ds(c * 16, 16)] = jnp.zeros((16,), jnp.float32)
        return carry

    lax.fori_loop(0, ZR, zrow, 0)
    zcopies = []
    for k in range(RPT // ZR):
        zcopies.append(pltpu.async_copy(
            rows.at[0, pl.ds(0, ZR)],
            acc.at[pl.ds(sid * RPT + k * ZR, ZR)], zsem))
    zcopies.append(pltpu.async_copy(
        rows.at[0, pl.ds(0, RPT - (RPT // ZR) * ZR)],
        acc.at[pl.ds(sid * RPT + (RPT // ZR) * ZR,
                     RPT - (RPT // ZR) * ZR)], zsem))

    wait_idx(0, isem0)
    fire_gather(0, 1, 1)
    for c in zcopies:
        c.wait()
    fire_gather(0, 0, 0)
    plsc.subcore_barrier()

    def group_body(g, s):
        s2 = 1 - s
        for k in range(G):
            buf = k % 2
            wait_gather(buf)
            pltpu.sync_copy(rows.at[buf], acc.at[didx.at[s, k]], add=True)
            if k == G - 2:
                wait_idx(s2, isems[s2])
            if k < G - 2:
                fire_gather(s, k + 2, buf)
            else:
                fire_gather(s2, k - (G - 2), buf)
        g2 = jnp.where(g + 2 >= NGROUP, g + 2 - NGROUP, g + 2)
        fire_idx(s, g2, isems[s])

    def outer(gp, carry):
        group_body(2 * gp, 0)
        group_body(2 * gp + 1, 1)
        return carry

    lax.fori_loop(0, NGROUP // 2, outer, 0)
    wait_gather(0)
    wait_gather(1)
    wait_idx(1, isem1)
    plsc.subcore_barrier()

    for k in range(RPT // ZROWS):
        r0 = sid * RPT + k * ZROWS
        pltpu.async_copy(acc.at[pl.ds(r0, ZROWS)],
                         out_hbm.at[cid, pl.ds(r0, ZROWS)], gsem0)
    for k in range(RPT // ZROWS):
        r0 = sid * RPT + k * ZROWS
        pltpu.make_async_copy(acc.at[pl.ds(r0, ZROWS)],
                              out_hbm.at[cid, pl.ds(r0, ZROWS)], gsem0).wait()


_agg = pl.kernel(
    _agg_body,
    out_type=jax.ShapeDtypeStruct((NC, NPAD, D), jnp.float32),
    mesh=plsc.VectorSubcoreMesh(core_axis_name="c", subcore_axis_name="s"),
    scratch_types=[
        pltpu.VMEM((2, G, CHUNK), jnp.int32),
        pltpu.VMEM((2, G, CHUNK), jnp.int32),
        pltpu.VMEM((NBUF, CHUNK, D), jnp.float32),
        pltpu.VMEM_SHARED((NPAD, D), jnp.float32),
        pltpu.SemaphoreType.DMA,
        pltpu.SemaphoreType.DMA,
        pltpu.SemaphoreType.DMA,
        pltpu.SemaphoreType.DMA,
        pltpu.SemaphoreType.DMA,
    ],
)


BLK = 2000


def _mlp_body(eps_ref, x_ref, p_ref, w1_ref, b1_ref, w2_ref, b2_ref, o_ref):
    hb = (1.0 + eps_ref[0]) * x_ref[...] + p_ref[0] + p_ref[1]
    t = jnp.dot(hb, w1_ref[...], preferred_element_type=jnp.float32) + b1_ref[...]
    t = jnp.maximum(t, 0.0)
    o_ref[...] = jnp.dot(t, w2_ref[...], preferred_element_type=jnp.float32) + b2_ref[...]


def _mlp(x, p, W1, b1, W2, b2, eps):
    return pl.pallas_call(
        _mlp_body,
        grid=(N // BLK,),
        in_specs=[
            pl.BlockSpec(memory_space=pltpu.SMEM),
            pl.BlockSpec((BLK, D), lambda i: (i, 0)),
            pl.BlockSpec((NC, BLK, D), lambda i: (0, i, 0)),
            pl.BlockSpec((D, D), lambda i: (0, 0)),
            pl.BlockSpec((1, D), lambda i: (0, 0)),
            pl.BlockSpec((D, D), lambda i: (0, 0)),
            pl.BlockSpec((1, D), lambda i: (0, 0)),
        ],
        out_specs=pl.BlockSpec((BLK, D), lambda i: (i, 0)),
        out_shape=jax.ShapeDtypeStruct((N, D), jnp.float32),
    )(eps.reshape(1), x, p, W1, b1.reshape(1, D), W2, b2.reshape(1, D))


def kernel(x, edge_index, W1_0, b1_0, W2_0, b2_0, eps_0,
           W1_1, b1_1, W2_1, b2_1, eps_1):
    h = x
    ei = edge_index.reshape(2, NW, NGROUP, G, CHUNK)
    for (W1, b1, W2, b2, eps) in ((W1_0, b1_0, W2_0, b2_0, eps_0),
                                  (W1_1, b1_1, W2_1, b2_1, eps_1)):
        p = _agg(h, ei)
        h = _mlp(h, p, W1, b1, W2, b2, eps)
    return h
